# baseline (speedup 1.0000x reference)
import functools

import jax
import jax.numpy as jnp
from jax import lax
from jax.experimental import pallas as pl
from jax.experimental.pallas import tpu as pltpu
from jax.experimental.pallas import tpu_sc as plsc

B, L, D, H = 1024, 200, 64, 8
N = B * L
NC, NS = 2, 16
NW = NC * NS
UE = 128
NB = B // UE
NUNIT = L * NB
UPW = NUNIT // NW


def _fused_sc(idx_f, v_f, cm_f, wb, W2, table):
    mesh = plsc.VectorSubcoreMesh(
        core_axis_name="c", subcore_axis_name="s", num_cores=NC, num_subcores=NS
    )

    @functools.partial(
        pl.kernel,
        out_type=[
            jax.ShapeDtypeStruct((L, 8, NB, 8, 128), jnp.float32),
            jax.ShapeDtypeStruct((L // 8, NB, 8, 128), jnp.float32),
        ],
        mesh=mesh,
        scratch_types=[
            pltpu.VMEM((UE,), jnp.int32),
            pltpu.VMEM((UE,), jnp.float32),
            pltpu.VMEM((UE,), jnp.float32),
            pltpu.VMEM((UE, D), jnp.float32),
            pltpu.VMEM((H, UE + 16), jnp.float32),
            pltpu.VMEM((D, UE), jnp.float32),
            pltpu.VMEM((UE,), jnp.float32),
            pltpu.VMEM((16,), jnp.float32),
            pltpu.VMEM((H, D), jnp.float32),
            pltpu.SemaphoreType.DMA,
        ],
        compiler_params=pltpu.CompilerParams(
            use_tc_tiling_on_sc=False, needs_layout_passes=False
        ),
    )
    def k(idx_hbm, v_hbm, cm_hbm, wb_hbm, w2_hbm, table_hbm,
          out_hbm, pm_hbm,
          idx_v, v_v, cm_v, rows_v, h_v, t_v, pm_v, wb_v, w2_v, sem):
        wid = lax.axis_index("s") * NC + lax.axis_index("c")
        pltpu.sync_copy(wb_hbm, wb_v)
        pltpu.sync_copy(w2_hbm, w2_v)
        iota16 = lax.iota(jnp.int32, 16)
        wb16 = wb_v[...]

        def unit(u, carry):
            uid = wid * UPW + u
            l = uid // NB
            b1 = uid % NB
            off = l * B + b1 * UE
            pltpu.sync_copy(idx_hbm.at[pl.ds(off, UE)], idx_v)
            pltpu.sync_copy(v_hbm.at[pl.ds(off, UE)], v_v)
            pltpu.sync_copy(cm_hbm.at[pl.ds(off, UE)], cm_v)
            pltpu.async_copy(table_hbm.at[idx_v], rows_v, sem).wait()

            for c in range(UE // 16):
                v16 = v_v[pl.ds(16 * c, 16)]
                cm16 = cm_v[pl.ds(16 * c, 16)]
                i16 = idx_v[pl.ds(16 * c, 16)]
                pm_v[pl.ds(16 * c, 16)] = jnp.minimum(
                    i16.astype(jnp.float32), 1.0)
                for j in range(H):
                    a = v16 * wb16[j] + wb16[H + j]
                    hm = (1.0 - 2.0 / (jnp.exp(a) + 1.0)) * cm16
                    h_v[j, pl.ds(16 * c, 16)] = hm

            def elem(e, c2):
                bspl = jnp.full((16,), e, jnp.int32)
                hs = [h_v[j, pl.ds(e, 16)][0] for j in range(H)]
                for q in range(D // 16):
                    acc = rows_v[e, pl.ds(16 * q, 16)]
                    for j in range(H):
                        acc = acc + hs[j] * w2_v[j, pl.ds(16 * q, 16)]
                    plsc.store_scatter(t_v, [iota16 + 16 * q, bspl], acc)
                return c2

            lax.fori_loop(0, UE, elem, 0, unroll=False)

            for t in range(8):
                pltpu.sync_copy(t_v.at[pl.ds(8 * t, 8)], out_hbm.at[l, t, b1])
            pltpu.sync_copy(pm_v, pm_hbm.at[l // 8, b1, l % 8])
            return carry

        lax.fori_loop(0, UPW, unit, 0, unroll=False)

    return k(idx_f, v_f, cm_f, wb, W2, table)


def kernel(value, var_id, category_mask, W1, b1, W2, emb_table):
    var_id = var_id.astype(jnp.int32)
    idx_f = var_id.T.reshape(N)
    v_f = value.astype(jnp.float32).T.reshape(N)
    cm_f = category_mask.astype(jnp.float32).T.reshape(N)
    wb = jnp.concatenate([2.0 * W1.reshape(H), 2.0 * b1.reshape(H)])
    out5, pm4 = _fused_sc(idx_f, v_f, cm_f, wb, W2, emb_table)
    sum_emb = out5.transpose(2, 4, 0, 1, 3).reshape(B, L, D)
    pm = pm4.transpose(1, 3, 0, 2).reshape(B, L)
    return (sum_emb, pm)


# speedup vs baseline: 1.1385x; 1.1385x over previous
import functools

import jax
import jax.numpy as jnp
from jax import lax
from jax.experimental import pallas as pl
from jax.experimental.pallas import tpu as pltpu
from jax.experimental.pallas import tpu_sc as plsc

B, L, D, H = 1024, 200, 64, 8
N = B * L
NC, NS = 2, 16
NW = NC * NS
UE = 128
NB = B // UE
NUNIT = L * NB
UPW = NUNIT // NW
NQ = D // 16


def _fused_sc(idx_f, v_f, cm_f, wb, W2, table):
    mesh = plsc.VectorSubcoreMesh(
        core_axis_name="c", subcore_axis_name="s", num_cores=NC, num_subcores=NS
    )

    @functools.partial(
        pl.kernel,
        out_type=[
            jax.ShapeDtypeStruct((L, 8, NB, 8, 128), jnp.float32),
            jax.ShapeDtypeStruct((L // 8, NB, 8, 128), jnp.float32),
        ],
        mesh=mesh,
        scratch_types=[
            pltpu.VMEM((UPW * UE,), jnp.int32),
            pltpu.VMEM((UPW * UE,), jnp.float32),
            pltpu.VMEM((UPW * UE,), jnp.float32),
            pltpu.VMEM((UE, D), jnp.float32),
            pltpu.VMEM((UE, D), jnp.float32),
            pltpu.VMEM((D, UE), jnp.float32),
            pltpu.VMEM((D, UE), jnp.float32),
            pltpu.VMEM((UE,), jnp.float32),
            pltpu.VMEM((UE,), jnp.float32),
            pltpu.VMEM((H, UE + 16), jnp.float32),
            pltpu.VMEM((16,), jnp.float32),
            pltpu.VMEM((H, D), jnp.float32),
            pltpu.SemaphoreType.DMA,
            pltpu.SemaphoreType.DMA,
            pltpu.SemaphoreType.DMA,
        ],
        compiler_params=pltpu.CompilerParams(
            use_tc_tiling_on_sc=False, needs_layout_passes=False
        ),
    )
    def k(idx_hbm, v_hbm, cm_hbm, wb_hbm, w2_hbm, table_hbm,
          out_hbm, pm_hbm,
          idx_s, v_s, cm_s, rows_a, rows_b, t_a, t_b, pm_a, pm_b,
          h_v, wb_v, w2_v, sem_in, sem_g, sem_out):
        wid = lax.axis_index("s") * NC + lax.axis_index("c")
        base = wid * (UPW * UE)
        pltpu.sync_copy(wb_hbm, wb_v)
        pltpu.sync_copy(w2_hbm, w2_v)
        pltpu.async_copy(idx_hbm.at[pl.ds(base, UPW * UE)], idx_s, sem_in)
        pltpu.async_copy(v_hbm.at[pl.ds(base, UPW * UE)], v_s, sem_in)
        pltpu.async_copy(cm_hbm.at[pl.ds(base, UPW * UE)], cm_s, sem_in)
        pltpu.make_async_copy(
            idx_hbm.at[pl.ds(base, UPW * UE)], idx_s, sem_in).wait()
        pltpu.make_async_copy(
            v_hbm.at[pl.ds(base, UPW * UE)], v_s, sem_in).wait()
        pltpu.make_async_copy(
            cm_hbm.at[pl.ds(base, UPW * UE)], cm_s, sem_in).wait()

        wb16 = wb_v[...]
        iota16 = lax.iota(jnp.int32, 16)
        w2r = [[w2_v[j, pl.ds(16 * q, 16)] for q in range(NQ)]
               for j in range(H)]

        def start_gather(u, rows):
            pltpu.async_copy(
                table_hbm.at[idx_s.at[pl.ds(u * UE, UE)]], rows, sem_g)

        def wait_gather(rows):
            pltpu.make_async_copy(table_hbm.at[pl.ds(0, UE)], rows,
                                  sem_g).wait()

        def drain_out(t_v, pm_v, l, b1):
            for t in range(8):
                pltpu.make_async_copy(
                    t_v.at[pl.ds(8 * t, 8)], out_hbm.at[l, t, b1],
                    sem_out).wait()
            pltpu.make_async_copy(
                pm_v, pm_hbm.at[l // 8, b1, l % 8], sem_out).wait()

        def run_unit(u, rows_cur, rows_nxt, t_v, pm_v):
            uid = wid * UPW + u
            l = uid // NB
            b1 = uid % NB
            wait_gather(rows_cur)
            @pl.when(u + 1 < UPW)
            def _():
                start_gather(u + 1, rows_nxt)
            @pl.when(u >= 2)
            def _():
                drain_out(t_v, pm_v, l, b1)

            for c in range(UE // 16):
                off = u * UE + 16 * c
                v16 = v_s[pl.ds(off, 16)]
                cm16 = cm_s[pl.ds(off, 16)]
                i16 = idx_s[pl.ds(off, 16)]
                pm_v[pl.ds(16 * c, 16)] = jnp.minimum(
                    i16.astype(jnp.float32), 1.0)
                for j in range(H):
                    a = v16 * wb16[j] + wb16[H + j]
                    hm = (1.0 - 2.0 / (jnp.exp(a) + 1.0)) * cm16
                    h_v[j, pl.ds(16 * c, 16)] = hm

            def elem(e, c2):
                bspl = jnp.full((16,), e, jnp.int32)
                hs = [h_v[j, pl.ds(e, 16)][0] for j in range(H)]
                for q in range(NQ):
                    acc = rows_cur[e, pl.ds(16 * q, 16)]
                    for j in range(H):
                        acc = acc + hs[j] * w2r[j][q]
                    plsc.store_scatter(t_v, [iota16 + 16 * q, bspl], acc)
                return c2

            lax.fori_loop(0, UE, elem, 0, unroll=2)

            for t in range(8):
                pltpu.async_copy(
                    t_v.at[pl.ds(8 * t, 8)], out_hbm.at[l, t, b1], sem_out)
            pltpu.async_copy(pm_v, pm_hbm.at[l // 8, b1, l % 8], sem_out)

        start_gather(0, rows_a)

        def pair(kk, carry):
            run_unit(2 * kk, rows_a, rows_b, t_a, pm_a)
            run_unit(2 * kk + 1, rows_b, rows_a, t_b, pm_b)
            return carry

        lax.fori_loop(0, UPW // 2, pair, 0, unroll=False)

        uidl = wid * UPW + (UPW - 2)
        drain_out(t_a, pm_a, uidl // NB, uidl % NB)
        drain_out(t_b, pm_b, (uidl + 1) // NB, (uidl + 1) % NB)

    return k(idx_f, v_f, cm_f, wb, W2, table)


def kernel(value, var_id, category_mask, W1, b1, W2, emb_table):
    var_id = var_id.astype(jnp.int32)
    idx_f = var_id.T.reshape(N)
    v_f = value.astype(jnp.float32).T.reshape(N)
    cm_f = category_mask.astype(jnp.float32).T.reshape(N)
    wb = jnp.concatenate([2.0 * W1.reshape(H), 2.0 * b1.reshape(H)])
    out5, pm4 = _fused_sc(idx_f, v_f, cm_f, wb, W2, emb_table)
    sum_emb = out5.transpose(2, 4, 0, 1, 3).reshape(B, L, D)
    pm = pm4.transpose(1, 3, 0, 2).reshape(B, L)
    return (sum_emb, pm)


# speedup vs baseline: 1.3917x; 1.2224x over previous
import functools

import jax
import jax.numpy as jnp
from jax import lax
from jax.experimental import pallas as pl
from jax.experimental.pallas import tpu as pltpu
from jax.experimental.pallas import tpu_sc as plsc

B, L, D, H = 1024, 200, 64, 8
N = B * L
NC, NS = 2, 16
NW = NC * NS
UE = 128
NB = B // UE
NUNIT = L * NB
UPW = NUNIT // NW
NQ = D // 16


def _cve_body(v_ref, cm_ref, w1_ref, b1_ref, w2t_ref, out_ref):
    for li in range(8):
        v = v_ref[pl.ds(li, 1), :]
        t = jnp.tanh(w1_ref[...] * v + b1_ref[...])
        t = t * cm_ref[pl.ds(li, 1), :]
        ve = lax.dot_general(
            w2t_ref[...], t, (((1,), (0,)), ((), ())),
            preferred_element_type=jnp.float32,
        )
        for b1 in range(NB):
            out_ref[li, :, b1] = ve[:, b1 * UE:(b1 + 1) * UE].reshape(8, 8, UE)


def _cve_tc(vT, cmT, w1c, b1c, W2T):
    return pl.pallas_call(
        _cve_body,
        grid=(L // 8,),
        in_specs=[
            pl.BlockSpec((8, B), lambda l: (l, 0)),
            pl.BlockSpec((8, B), lambda l: (l, 0)),
            pl.BlockSpec((H, 1), lambda l: (0, 0)),
            pl.BlockSpec((H, 1), lambda l: (0, 0)),
            pl.BlockSpec((D, H), lambda l: (0, 0)),
        ],
        out_specs=pl.BlockSpec((8, 8, NB, 8, UE), lambda l: (l, 0, 0, 0, 0)),
        out_shape=jax.ShapeDtypeStruct((L, 8, NB, 8, UE), jnp.float32),
    )(vT, cmT, w1c, b1c, W2T)


def _sc_gather_add(idx_f, ve5, table):
    mesh = plsc.VectorSubcoreMesh(
        core_axis_name="c", subcore_axis_name="s", num_cores=NC, num_subcores=NS
    )

    @functools.partial(
        pl.kernel,
        out_type=[
            jax.ShapeDtypeStruct((L, 8, NB, 8, 128), jnp.float32),
            jax.ShapeDtypeStruct((L // 8, NB, 8, 128), jnp.float32),
        ],
        mesh=mesh,
        scratch_types=[
            pltpu.VMEM((UPW * UE,), jnp.int32),
            pltpu.VMEM((UE, D), jnp.float32),
            pltpu.VMEM((UE, D), jnp.float32),
            pltpu.VMEM((3, D, UE), jnp.float32),
            pltpu.VMEM((UE,), jnp.float32),
            pltpu.VMEM((UE,), jnp.float32),
            pltpu.SemaphoreType.DMA,
            pltpu.SemaphoreType.DMA,
            pltpu.SemaphoreType.DMA,
            pltpu.SemaphoreType.DMA,
        ],
        compiler_params=pltpu.CompilerParams(
            use_tc_tiling_on_sc=False, needs_layout_passes=False
        ),
    )
    def k(idx_hbm, ve_hbm, table_hbm, out_hbm, pm_hbm,
          idx_s, rows_a, rows_b, t_v, pm_a, pm_b,
          sem_in, sem_g, sem_ve, sem_out):
        wid = lax.axis_index("s") * NC + lax.axis_index("c")
        base = wid * (UPW * UE)
        pltpu.sync_copy(idx_hbm.at[pl.ds(base, UPW * UE)], idx_s)
        iota16 = lax.iota(jnp.int32, 16)

        def unit_lb(u):
            uid = wid * UPW + u
            return uid // NB, uid % NB

        def start_gather(u, rows):
            pltpu.async_copy(
                table_hbm.at[idx_s.at[pl.ds(u * UE, UE)]], rows, sem_g)

        def wait_gather(rows):
            pltpu.make_async_copy(table_hbm.at[pl.ds(0, UE)], rows,
                                  sem_g).wait()

        def start_ve(u, slot):
            l, b1 = unit_lb(u)
            for t in range(8):
                pltpu.async_copy(
                    ve_hbm.at[l, t, b1], t_v.at[slot, pl.ds(8 * t, 8)],
                    sem_ve)

        def wait_ve(slot):
            for t in range(8):
                pltpu.make_async_copy(
                    ve_hbm.at[0, t, 0], t_v.at[slot, pl.ds(8 * t, 8)],
                    sem_ve).wait()

        def start_out(u, slot, pm_v):
            l, b1 = unit_lb(u)
            for t in range(8):
                pltpu.async_copy(
                    t_v.at[slot, pl.ds(8 * t, 8)], out_hbm.at[l, t, b1],
                    sem_out)
            pltpu.async_copy(pm_v, pm_hbm.at[l // 8, b1, l % 8], sem_out)

        def drain_out(u, slot, pm_v):
            l, b1 = unit_lb(u)
            for t in range(8):
                pltpu.make_async_copy(
                    t_v.at[slot, pl.ds(8 * t, 8)], out_hbm.at[l, t, b1],
                    sem_out).wait()
            pltpu.make_async_copy(
                pm_v, pm_hbm.at[l // 8, b1, l % 8], sem_out).wait()

        def run_unit(u, rows_cur, rows_nxt, pm_v):
            slot = u % 3
            wait_gather(rows_cur)
            wait_ve(slot)
            @pl.when(u + 1 < UPW)
            def _():
                start_gather(u + 1, rows_nxt)
            @pl.when(u >= 1)
            def _():
                drain_out(u - 1, (u + 2) % 3, pm_v)
            @pl.when(u + 2 < UPW)
            def _():
                start_ve(u + 2, (u + 2) % 3)

            for c in range(UE // 16):
                i16 = idx_s[pl.ds(u * UE + 16 * c, 16)]
                pm_v[pl.ds(16 * c, 16)] = jnp.minimum(
                    i16.astype(jnp.float32), 1.0)

            def elem(e, c2):
                bspl = jnp.full((16,), e, jnp.int32)
                for q in range(NQ):
                    acc = rows_cur[e, pl.ds(16 * q, 16)]
                    plsc.addupdate_scatter(
                        t_v.at[slot], [iota16 + 16 * q, bspl], acc)
                return c2

            lax.fori_loop(0, UE, elem, 0, unroll=4)
            start_out(u, slot, pm_v)

        start_gather(0, rows_a)
        start_ve(0, 0)
        start_ve(1, 1)

        def pair(kk, carry):
            run_unit(2 * kk, rows_a, rows_b, pm_a)
            run_unit(2 * kk + 1, rows_b, rows_a, pm_b)
            return carry

        lax.fori_loop(0, UPW // 2, pair, 0, unroll=False)
        drain_out(UPW - 1, (UPW - 1) % 3, pm_b)

    return k(idx_f, ve5, table)


def kernel(value, var_id, category_mask, W1, b1, W2, emb_table):
    var_id = var_id.astype(jnp.int32)
    idx_f = var_id.T.reshape(N)
    vT = value.astype(jnp.float32).T
    cmT = category_mask.astype(jnp.float32).T
    ve5 = _cve_tc(vT, cmT, W1.reshape(H, 1), b1.reshape(H, 1), W2.T)
    out5, pm4 = _sc_gather_add(idx_f, ve5, emb_table)
    sum_emb = out5.transpose(2, 4, 0, 1, 3).reshape(B, L, D)
    pm = pm4.transpose(1, 3, 0, 2).reshape(B, L)
    return (sum_emb, pm)
